# baseline (speedup 1.0000x reference)
import functools

import jax
import jax.numpy as jnp
from jax import lax
from jax.experimental import pallas as pl
from jax.experimental.pallas import tpu as pltpu
from jax.experimental.pallas import tpu_sc as plsc

NUM_USERS = 1000000
NUM_ITEMS = 100000
NUM_FEATURES = 100000
EMBED_DIM = 128
BATCH = 16384
N_FIELDS = 26

NC = 2
NS = 16
NW = NC * NS

CHUNK = 256
NBUF = 3
BAT_W = BATCH // NW
HALVES = BAT_W // CHUNK
U_OFF = 0
I_OFF = BAT_W
F_OFF = 2 * BAT_W
IDX_WORDS = (2 + N_FIELDS) * BAT_W
UC = BAT_W // CHUNK
NV = 2 * UC + N_FIELDS * HALVES
PREFIX = 6


def _lookup_kernel(user_ids, item_ids, feature_ids_t,
                   user_table, item_table, feature_table):
    mesh = plsc.VectorSubcoreMesh(core_axis_name="c", subcore_axis_name="s")

    @functools.partial(
        pl.kernel,
        mesh=mesh,
        out_type=(
            jax.ShapeDtypeStruct((BATCH, EMBED_DIM), jnp.float32),
            jax.ShapeDtypeStruct((BATCH, EMBED_DIM), jnp.float32),
            jax.ShapeDtypeStruct((N_FIELDS * BATCH, EMBED_DIM), jnp.float32),
        ),
        scratch_types=[
            pltpu.VMEM((IDX_WORDS,), jnp.int32),
            pltpu.VMEM((CHUNK, EMBED_DIM), jnp.float32),
            pltpu.VMEM((CHUNK, EMBED_DIM), jnp.float32),
            pltpu.VMEM((CHUNK, EMBED_DIM), jnp.float32),
            pltpu.SemaphoreType.DMA,
            pltpu.SemaphoreType.DMA,
            pltpu.SemaphoreType.DMA,
            pltpu.SemaphoreType.DMA,
            pltpu.SemaphoreType.DMA,
            pltpu.SemaphoreType.DMA,
            pltpu.SemaphoreType.DMA,
        ],
    )
    def k(uids, iids, fids_t, utab, itab, ftab, uout, iout, fout,
          idx_v, rows0, rows1, rows2, g0, g1, g2, o0, o1, o2, psem):
        wid = lax.axis_index("s") * NC + lax.axis_index("c")
        rows = (rows0, rows1, rows2)
        gsem = (g0, g1, g2)
        osem = (o0, o1, o2)
        wbase = wid * BAT_W

        def pre_descs():
            descs = [
                (uids.at[pl.ds(wbase, BAT_W)],
                 idx_v.at[pl.ds(U_OFF, BAT_W)]),
                (iids.at[pl.ds(wbase, BAT_W)],
                 idx_v.at[pl.ds(I_OFF, BAT_W)]),
            ]
            for f in range(N_FIELDS):
                descs.append((fids_t.at[f, pl.ds(wbase, BAT_W)],
                              idx_v.at[pl.ds(F_OFF + f * BAT_W, BAT_W)]))
            return descs

        for src, dst in pre_descs():
            pltpu.async_copy(src, dst, psem)

        def wait_prefetch(n_slices):
            src, dst = pre_descs()[0]
            for _ in range(n_slices):
                pltpu.make_async_copy(src, dst, psem).wait()

        def fspec(fc):
            f = fc // HALVES
            h = fc % HALVES
            return (ftab, idx_v.at[pl.ds(F_OFF + fc * CHUNK, CHUNK)],
                    fout.at[pl.ds(f * BATCH + wbase + h * CHUNK, CHUNK)])

        def spec(vc):
            if vc < UC:
                return (utab, idx_v.at[pl.ds(U_OFF + vc * CHUNK, CHUNK)],
                        uout.at[pl.ds(wbase + vc * CHUNK, CHUNK)])
            if vc < 2 * UC:
                c = vc - UC
                return (itab, idx_v.at[pl.ds(I_OFF + c * CHUNK, CHUNK)],
                        iout.at[pl.ds(wbase + c * CHUNK, CHUNK)])
            return fspec(vc - 2 * UC)

        def start_g(sp, b):
            tab, idx, _ = sp
            pltpu.async_copy(tab.at[idx], rows[b], gsem[b])

        def wait_g(sp, b):
            tab, idx, _ = sp
            pltpu.make_async_copy(tab.at[idx], rows[b], gsem[b]).wait()

        def start_o(sp, b):
            _, _, out = sp
            pltpu.async_copy(rows[b], out, osem[b])

        def wait_o(sp, b):
            _, _, out = sp
            pltpu.make_async_copy(rows[b], out, osem[b]).wait()

        wait_prefetch(2)
        start_g(spec(0), 0)
        start_g(spec(1), 1)

        for vc in range(PREFIX):
            b = vc % NBUF
            if vc == 2:
                wait_prefetch(N_FIELDS)
            wait_g(spec(vc), b)
            start_o(spec(vc), b)
            if vc > 0:
                wait_o(spec(vc - 1), (vc - 1) % NBUF)
            start_g(spec(vc + 2), (vc + 2) % NBUF)

        ngroups = (NV - PREFIX - 2) // NBUF

        def body(g, carry):
            for b in range(NBUF):
                fc = (PREFIX - 2 * UC) + g * NBUF + b
                wait_g(fspec(fc), b)
                start_o(fspec(fc), b)
                wait_o(fspec(fc - 1), (b + 2) % NBUF)
                start_g(fspec(fc + 2), (b + 2) % NBUF)
            return carry

        lax.fori_loop(0, ngroups, body, 0)

        for vc in range(PREFIX + ngroups * NBUF, NV):
            b = vc % NBUF
            wait_g(spec(vc), b)
            start_o(spec(vc), b)
            wait_o(spec(vc - 1), (vc - 1) % NBUF)
        wait_o(spec(NV - 1), (NV - 1) % NBUF)

    return k(user_ids, item_ids, feature_ids_t,
             user_table, item_table, feature_table)


def kernel(user_ids, item_ids, feature_ids, user_table, item_table,
           feature_table):
    user_emb, item_emb, feat_fmajor = _lookup_kernel(
        user_ids, item_ids, jnp.transpose(feature_ids),
        user_table, item_table, feature_table)
    feat_emb = feat_fmajor.reshape(N_FIELDS, BATCH, EMBED_DIM)
    return (user_emb, item_emb, jnp.transpose(feat_emb, (1, 0, 2)))
